# baseline (speedup 1.0000x reference)
import functools

import jax
import jax.numpy as jnp
from jax import lax
from jax.experimental import pallas as pl
from jax.experimental.pallas import tpu as pltpu
from jax.experimental.pallas import tpu_sc as plsc

SIGMA = 0.4
N_IN = 40962
N_OUT = 163842
C = 128
K = 7
NW = 32
G = 48
CPW = 108
RPW = G * CPW
STRIDE = 5121
LAST_START = N_OUT - RPW
GK = G * K
WIN = GK + 8
IDX_MINOR = 112
NSEG = GK // IDX_MINOR
FLAT = N_OUT * K
FLATP = ((FLAT + 7) // 8) * 8 + 8


def _sc_body(x_hbm, idx_hbm, dm_hbm, mk_hbm, out_hbm,
             idx_v, dm_v, mk_v, idx_f, gath_v, outb_v,
             sem_in0, sem_in1, sem_g0, sem_g1, sem_o0, sem_o1):
    cid = lax.axis_index("c")
    sid = lax.axis_index("s")
    wid = sid * 2 + cid
    start = jnp.minimum(wid * STRIDE, LAST_START)
    sem_in = [sem_in0, sem_in1]
    sem_g = [sem_g0, sem_g1]
    sem_o = [sem_o0, sem_o1]
    c1 = -1.0 / (2.0 * SIGMA * SIGMA)
    iota = lax.iota(jnp.int32, 16)
    iota7 = iota * K

    def win_off(c):
        e0 = (start + c * G) * K
        e0a = pl.multiple_of((e0 // 8) * 8, 8)
        return e0a, e0 - e0a

    def in_descs(c, b):
        e0a, _ = win_off(c)
        src = pl.ds(e0a, WIN)
        return [
            pltpu.make_async_copy(idx_hbm.at[src], idx_v.at[b], sem_in[b]),
            pltpu.make_async_copy(dm_hbm.at[src], dm_v.at[b], sem_in[b]),
            pltpu.make_async_copy(mk_hbm.at[src], mk_v.at[b], sem_in[b]),
        ]

    def repack_idx(c, b):
        _, d = win_off(c)
        for t in range(GK // 16):
            v = plsc.load_gather(idx_v.at[b], [d + t * 16 + iota])
            idx_f[b, t // 7, pl.ds((t % 7) * 16, 16)] = v

    def gath_descs(b):
        return [
            pltpu.make_async_copy(
                x_hbm.at[idx_f.at[b, j]],
                gath_v.at[b, pl.ds(j * IDX_MINOR, IDX_MINOR)],
                sem_g[b])
            for j in range(NSEG)
        ]

    def out_desc(c, b):
        return pltpu.make_async_copy(
            outb_v.at[b],
            out_hbm.at[pl.ds((start + c * G) * C, G * C)],
            sem_o[b])

    def compute(c, b):
        _, d = win_off(c)

        @plsc.parallel_loop(0, G // 16)
        def group_body(j):
            g0 = j * 16
            off = d + g0 * K + iota7
            wks = []
            for k in range(K):
                dd = plsc.load_gather(dm_v.at[b], [off + k])
                m = plsc.load_gather(mk_v.at[b], [off + k])
                wks.append(jnp.exp(dd * dd * c1) * m)
            wsum = wks[0]
            for k in range(1, K):
                wsum = wsum + wks[k]
            inv = 1.0 / jnp.maximum(wsum, 1e-8)
            swks = [wk * inv for wk in wks]
            for r in range(16):
                ws = [swks[k][r] for k in range(K)]
                base = (g0 + r) * K
                accs = []
                for cc in range(C // 16):
                    csl = pl.ds(cc * 16, 16)
                    p = [ws[k] * gath_v[b, base + k, csl] for k in range(K)]
                    s01 = p[0] + p[1]
                    s23 = p[2] + p[3]
                    s45 = p[4] + p[5]
                    accs.append((s01 + s23) + (s45 + p[6]))
                obase = (g0 + r) * C
                for cc in range(C // 16):
                    outb_v[b, pl.ds(obase + cc * 16, 16)] = accs[cc]

    for dsc in in_descs(0, 0):
        dsc.start()
    for dsc in in_descs(0, 0):
        dsc.wait()
    repack_idx(0, 0)
    for dsc in gath_descs(0):
        dsc.start()
    for dsc in in_descs(1, 1):
        dsc.start()

    def pair_body(it, carry):
        c0 = it * 2
        for b in range(2):
            c = c0 + b
            nb = 1 - b
            @pl.when(c + 1 < CPW)
            def _():
                for dsc in in_descs(c + 1, nb):
                    dsc.wait()
                repack_idx(c + 1, nb)
                for dsc in gath_descs(nb):
                    dsc.start()

            for dsc in gath_descs(b):
                dsc.wait()

            @pl.when(c >= 2)
            def _():
                out_desc(c - 2, b).wait()

            compute(c, b)
            out_desc(c, b).start()

            @pl.when(c + 2 < CPW)
            def _():
                for dsc in in_descs(c + 2, b):
                    dsc.start()
        return carry

    lax.fori_loop(0, CPW // 2, pair_body, 0)
    out_desc(CPW - 2, 0).wait()
    out_desc(CPW - 1, 1).wait()


def kernel(x, cand_idx, cand_mask, delta):
    x2 = x.reshape(N_IN, C)
    pad = FLATP - FLAT

    def flat(a):
        return jnp.pad(a.reshape(FLAT), (0, pad))

    sc_fn = functools.partial(
        pl.kernel,
        mesh=plsc.VectorSubcoreMesh(core_axis_name="c", subcore_axis_name="s"),
        out_type=jax.ShapeDtypeStruct((N_OUT * C,), jnp.float32),
        scratch_types=[
            pltpu.VMEM((2, WIN), jnp.int32),
            pltpu.VMEM((2, WIN), jnp.float32),
            pltpu.VMEM((2, WIN), jnp.float32),
            pltpu.VMEM((2, NSEG, IDX_MINOR), jnp.int32),
            pltpu.VMEM((2, GK, C), jnp.float32),
            pltpu.VMEM((2, G * C), jnp.float32),
            pltpu.SemaphoreType.DMA,
            pltpu.SemaphoreType.DMA,
            pltpu.SemaphoreType.DMA,
            pltpu.SemaphoreType.DMA,
            pltpu.SemaphoreType.DMA,
            pltpu.SemaphoreType.DMA,
        ],
        compiler_params=pltpu.CompilerParams(
            use_tc_tiling_on_sc=False, needs_layout_passes=False),
    )(_sc_body)
    out = sc_fn(x2, flat(cand_idx.astype(jnp.int32)), flat(delta),
                flat(cand_mask))
    return out.reshape(1, N_OUT, C)


# speedup vs baseline: 1.0448x; 1.0448x over previous
import functools

import jax
import jax.numpy as jnp
from jax import lax
from jax.experimental import pallas as pl
from jax.experimental.pallas import tpu as pltpu
from jax.experimental.pallas import tpu_sc as plsc

SIGMA = 0.4
N_IN = 40962
N_INP = 40968
N_OUT = 163842
NCOL = 163968
C = 128
K = 7
NW = 32
G = 48
CPW = 108
RPW = G * CPW
STRIDE = 5121
LAST_START = N_OUT - RPW
GK = G * K
WIN = G + 8
IDX_MINOR = 112
NSEG = GK // IDX_MINOR


def _sc_body(x_hbm, idx_hbm, dm_hbm, mk_hbm, out_hbm,
             idx_v, dm_v, mk_v, idx_f, gath_v, outb_v,
             sem_in0, sem_in1, sem_g0, sem_g1, sem_o0, sem_o1):
    cid = lax.axis_index("c")
    sid = lax.axis_index("s")
    wid = sid * 2 + cid
    start = jnp.minimum(wid * STRIDE, LAST_START)
    sem_in = [sem_in0, sem_in1]
    sem_g = [sem_g0, sem_g1]
    sem_o = [sem_o0, sem_o1]
    c1 = -1.0 / (2.0 * SIGMA * SIGMA)
    iota = lax.iota(jnp.int32, 16)

    def win_off(c):
        col0 = start + c * G
        col0a = pl.multiple_of((col0 // 8) * 8, 8)
        return col0a, col0 - col0a

    def in_descs(c, b):
        col0a, _ = win_off(c)
        src = pl.ds(col0a, WIN)
        return [
            pltpu.make_async_copy(idx_hbm.at[:, src], idx_v.at[b], sem_in[b]),
            pltpu.make_async_copy(dm_hbm.at[:, src], dm_v.at[b], sem_in[b]),
            pltpu.make_async_copy(mk_hbm.at[:, src], mk_v.at[b], sem_in[b]),
        ]

    def repack_idx(c, b):
        _, d = win_off(c)
        for k in range(K):
            rows = jnp.full((16,), k, jnp.int32)
            for s in range(G // 16):
                v = plsc.load_gather(idx_v.at[b], [rows, d + s * 16 + iota])
                p = k * G + s * 16
                idx_f[b, p // IDX_MINOR, pl.ds(p % IDX_MINOR, 16)] = v

    def gath_descs(b):
        return [
            pltpu.make_async_copy(
                x_hbm.at[idx_f.at[b, j]],
                gath_v.at[b, pl.ds(j * IDX_MINOR, IDX_MINOR)],
                sem_g[b])
            for j in range(NSEG)
        ]

    def out_desc(c, b):
        return pltpu.make_async_copy(
            outb_v.at[b], out_hbm.at[pl.ds(start + c * G, G)], sem_o[b])

    def compute(c, b):
        _, d = win_off(c)

        @plsc.parallel_loop(0, G // 16)
        def group_body(j):
            g0 = j * 16
            cols = d + g0 + iota
            wks = []
            for k in range(K):
                rows = jnp.full((16,), k, jnp.int32)
                dd = plsc.load_gather(dm_v.at[b], [rows, cols])
                m = plsc.load_gather(mk_v.at[b], [rows, cols])
                wks.append(jnp.exp(dd * dd * c1) * m)
            wsum = wks[0]
            for k in range(1, K):
                wsum = wsum + wks[k]
            inv = 1.0 / jnp.maximum(wsum, 1e-8)
            swks = [wk * inv for wk in wks]
            for r in range(16):
                ws = [swks[k][r] for k in range(K)]
                g = g0 + r
                accs = []
                for cc in range(C // 16):
                    csl = pl.ds(cc * 16, 16)
                    p = [ws[k] * gath_v[b, k * G + g, csl] for k in range(K)]
                    s01 = p[0] + p[1]
                    s23 = p[2] + p[3]
                    s45 = p[4] + p[5]
                    accs.append((s01 + s23) + (s45 + p[6]))
                for cc in range(C // 16):
                    outb_v[b, g, pl.ds(cc * 16, 16)] = accs[cc]

    for dsc in in_descs(0, 0):
        dsc.start()
    for dsc in in_descs(0, 0):
        dsc.wait()
    repack_idx(0, 0)
    for dsc in gath_descs(0):
        dsc.start()
    for dsc in in_descs(1, 1):
        dsc.start()

    def pair_body(it, carry):
        c0 = it * 2
        for b in range(2):
            c = c0 + b
            nb = 1 - b
            @pl.when(c + 1 < CPW)
            def _():
                for dsc in in_descs(c + 1, nb):
                    dsc.wait()
                repack_idx(c + 1, nb)
                for dsc in gath_descs(nb):
                    dsc.start()

            for dsc in gath_descs(b):
                dsc.wait()

            @pl.when(c >= 2)
            def _():
                out_desc(c - 2, b).wait()

            compute(c, b)
            out_desc(c, b).start()

            @pl.when(c + 2 < CPW)
            def _():
                for dsc in in_descs(c + 2, b):
                    dsc.start()
        return carry

    lax.fori_loop(0, CPW // 2, pair_body, 0)
    out_desc(CPW - 2, 0).wait()
    out_desc(CPW - 1, 1).wait()


def kernel(x, cand_idx, cand_mask, delta):
    xp = jnp.pad(x.reshape(N_IN, C), ((0, N_INP - N_IN), (0, 0)))

    def soa(a):
        return jnp.pad(a.T, ((0, 8 - K), (0, NCOL - N_OUT)))

    sc_fn = functools.partial(
        pl.kernel,
        mesh=plsc.VectorSubcoreMesh(core_axis_name="c", subcore_axis_name="s"),
        out_type=jax.ShapeDtypeStruct((N_OUT, C), jnp.float32),
        scratch_types=[
            pltpu.VMEM((2, 8, WIN), jnp.int32),
            pltpu.VMEM((2, 8, WIN), jnp.float32),
            pltpu.VMEM((2, 8, WIN), jnp.float32),
            pltpu.VMEM((2, NSEG, IDX_MINOR), jnp.int32),
            pltpu.VMEM((2, GK, C), jnp.float32),
            pltpu.VMEM((2, G, C), jnp.float32),
            pltpu.SemaphoreType.DMA,
            pltpu.SemaphoreType.DMA,
            pltpu.SemaphoreType.DMA,
            pltpu.SemaphoreType.DMA,
            pltpu.SemaphoreType.DMA,
            pltpu.SemaphoreType.DMA,
        ],
        compiler_params=pltpu.CompilerParams(
            use_tc_tiling_on_sc=False, needs_layout_passes=False),
    )(_sc_body)
    out = sc_fn(xp, soa(cand_idx.astype(jnp.int32)), soa(delta),
                soa(cand_mask))
    return out.reshape(1, N_OUT, C)
